# baseline (speedup 1.0000x reference)
import math
from functools import partial

import jax
import jax.numpy as jnp
import numpy as np
from jax.experimental import pallas as pl

_E = 8
_NFREQ = 3
_FCH = 4 + 8 * _NFREQ
_CFG = [(16, 3, 8), (14, 8, 16)]


def _fourier_pooled_const(h, w, ps):
    hp, wp = h // ps, w // ps
    ys = (np.arange(h, dtype=np.float32) + 0.5) / h
    xs = (np.arange(w, dtype=np.float32) + 0.5) / w
    pys = ((np.arange(h) // ps).astype(np.float32) + 0.5) / hp
    pxs = ((np.arange(w) // ps).astype(np.float32) + 0.5) / wp
    Y = np.broadcast_to(ys[:, None], (h, w))
    Xc = np.broadcast_to(xs[None, :], (h, w))
    PY = np.broadcast_to(pys[:, None], (h, w))
    PX = np.broadcast_to(pxs[None, :], (h, w))
    feats = [Y, Xc, PY, PX]
    for f in range(_NFREQ):
        fr = (2.0 ** f) * np.pi
        for m in (Y, Xc, PY, PX):
            feats.append(np.sin(fr * m))
            feats.append(np.cos(fr * m))
    four = np.stack(feats, 0)
    pm = four.reshape(_FCH, hp, ps, wp, ps).mean(axis=(2, 4))
    return jnp.asarray(pm.transpose(1, 2, 0).reshape(hp * wp, _FCH))


def _pool_body(x_ref, o_ref):
    o_ref[...] = jnp.mean(x_ref[...], axis=2)


def _router_body(pooled_ref, wg_ref, bg_ref, g_ref, aux_ref, *, cap, n):
    pooled = pooled_ref[...]
    logits = jnp.dot(pooled, wg_ref[...], preferred_element_type=jnp.float32)
    logits = logits + bg_ref[...]
    m = jnp.max(logits, axis=1, keepdims=True)
    ex = jnp.exp(logits - m)
    p = ex / jnp.sum(ex, axis=1, keepdims=True)
    gate = jnp.max(p, axis=1)
    idx = jnp.argmax(p, axis=1)
    oh = (jax.lax.broadcasted_iota(jnp.int32, (n, _E), 1) == idx[:, None])
    oh = oh.astype(jnp.float32)
    nb = n // 128
    oh3 = oh.reshape(nb, 128, _E)
    rowsum = jnp.sum(oh3, axis=1)
    li = jax.lax.broadcasted_iota(jnp.int32, (nb, nb), 0)
    lj = jax.lax.broadcasted_iota(jnp.int32, (nb, nb), 1)
    lstrict = (lj < li).astype(jnp.float32)
    prefix = jnp.dot(lstrict, rowsum, preferred_element_type=jnp.float32)
    ui = jax.lax.broadcasted_iota(jnp.int32, (128, 128), 0)
    uj = jax.lax.broadcasted_iota(jnp.int32, (128, 128), 1)
    upper = (ui <= uj).astype(jnp.float32)
    gate3 = gate.reshape(nb, 128)
    outs = []
    for e in range(_E):
        ohe = oh3[:, :, e]
        intra = jnp.dot(ohe, upper, preferred_element_type=jnp.float32)
        rank = prefix[:, e][:, None] + intra
        keep = (rank <= cap).astype(jnp.float32)
        outs.append(ohe * keep * gate3)
    g = jnp.stack(outs, axis=2).reshape(n, _E)
    g_ref[...] = g
    psum = jnp.sum(p, axis=0)
    cnt = jnp.sum(rowsum, axis=0)
    aux_ref[...] = (_E * jnp.sum(cnt * psum) / float(n * n)).reshape(1, 1)


def _conv_body(x_ref, g_ref, w_ref, b_ref, o_ref, *, cout):
    y = jnp.dot(x_ref[...], w_ref[...], preferred_element_type=jnp.float32)
    y = jnp.maximum(y + b_ref[...], 0.0)
    g = g_ref[...]
    acc = y[:, 0:cout] * g[:, 0:1]
    for e in range(1, _E):
        acc = acc + y[:, e * cout:(e + 1) * cout] * g[:, e:e + 1]
    o_ref[...] = acc


def _head_mm_body(x_ref, w_ref, b_ref, o_ref):
    o_ref[...] = jnp.dot(x_ref[...], w_ref[...],
                         preferred_element_type=jnp.float32) + b_ref[...]


def _pce_layer(x, ps, wconv, bconv, wg, bg, cout, tbp, tbc):
    b, c, h, w = x.shape
    hp, wp = h // ps, w // ps
    npatch = hp * wp
    n = b * npatch
    ppix = ps * ps
    cap = int(math.ceil(n / _E * 1.25))
    xp = x.reshape(b, c, hp, ps, wp, ps).transpose(0, 2, 4, 1, 3, 5)
    xp = xp.reshape(n, c, ps, ps)

    pooled_x = pl.pallas_call(
        _pool_body,
        grid=(n // tbp,),
        in_specs=[pl.BlockSpec((tbp, c, ppix), lambda i: (i, 0, 0))],
        out_specs=pl.BlockSpec((tbp, c), lambda i: (i, 0)),
        out_shape=jax.ShapeDtypeStruct((n, c), jnp.float32),
    )(xp.reshape(n, c, ppix))
    fp = jnp.tile(_fourier_pooled_const(h, w, ps), (b, 1))
    pooled = jnp.concatenate([pooled_x, fp], axis=1)

    g, aux = pl.pallas_call(
        partial(_router_body, cap=cap, n=n),
        out_shape=(
            jax.ShapeDtypeStruct((n, _E), jnp.float32),
            jax.ShapeDtypeStruct((1, 1), jnp.float32),
        ),
    )(pooled, wg, bg.reshape(1, _E))

    k = c * 9
    xpad = jnp.pad(xp, ((0, 0), (0, 0), (1, 1), (1, 1)))
    cols = jnp.stack(
        [xpad[:, :, dy:dy + ps, dx:dx + ps] for dy in range(3) for dx in range(3)],
        axis=2,
    )
    xcol = cols.transpose(0, 3, 4, 1, 2).reshape(n * ppix, k)
    wall = wconv.transpose(2, 3, 4, 0, 1).reshape(k, _E * cout)
    ball = bconv.reshape(1, _E * cout)
    grow = jnp.repeat(g, ppix, axis=0)

    rows = tbc * ppix
    y = pl.pallas_call(
        partial(_conv_body, cout=cout),
        grid=(n // tbc,),
        in_specs=[
            pl.BlockSpec((rows, k), lambda i: (i, 0)),
            pl.BlockSpec((rows, _E), lambda i: (i, 0)),
            pl.BlockSpec((k, _E * cout), lambda i: (0, 0)),
            pl.BlockSpec((1, _E * cout), lambda i: (0, 0)),
        ],
        out_specs=pl.BlockSpec((rows, cout), lambda i: (i, 0)),
        out_shape=jax.ShapeDtypeStruct((n * ppix, cout), jnp.float32),
    )(xcol, grow, wall, ball)

    y = y.reshape(b, hp, wp, ps, ps, cout).transpose(0, 5, 1, 3, 2, 4)
    return y.reshape(b, cout, h, w), aux[0, 0]


def kernel(X, params):
    x = X
    aux = jnp.float32(0.0)
    tbs = [(112, 28), (256, 32)]
    for li, (ps, cin, cout) in enumerate(_CFG):
        x, aux = _pce_layer(
            x, ps,
            params['Wconv%d' % li], params['bconv%d' % li],
            params['Wg%d' % li], params['bg%d' % li],
            cout, tbs[li][0], tbs[li][1],
        )
    b, c, h, w = x.shape
    xf = x.reshape(b, c, 8, h // 8, 8, w // 8).transpose(0, 1, 2, 4, 3, 5)
    xf = xf.reshape(b, c * 64, (h // 8) * (w // 8))
    npix = (h // 8) * (w // 8)
    pooled = pl.pallas_call(
        _pool_body,
        grid=(b // 8,),
        in_specs=[pl.BlockSpec((8, c * 64, npix), lambda i: (i, 0, 0))],
        out_specs=pl.BlockSpec((8, c * 64), lambda i: (i, 0)),
        out_shape=jax.ShapeDtypeStruct((b, c * 64), jnp.float32),
    )(xf)
    logits = pl.pallas_call(
        _head_mm_body,
        out_shape=jax.ShapeDtypeStruct((b, 1000), jnp.float32),
    )(pooled, params['Wlin'], params['blin'].reshape(1, 1000))
    return logits, aux
